# Initial kernel scaffold; baseline (speedup 1.0000x reference)
import jax
import jax.numpy as jnp
from jax.experimental import pallas as pl


def kernel(x, edge_index, batch, W1, b1, a1, W2, b2, a2):
    raise NotImplementedError("write your pallas kernel here")


# speedup vs baseline: 14.8689x; 14.8689x over previous
import functools

import jax
import jax.numpy as jnp
from jax import lax
from jax.experimental import pallas as pl
from jax.experimental.pallas import tpu as pltpu
from jax.experimental.pallas import tpu_sc as plsc

N = 10000
E = 320000
D = 128
G = 16

NC, NS = 2, 16
NW = NC * NS
C = 128
EWP = 10240
J = EWP // C
JH = J // 2
NPAD = 10240
RPT = NPAD // NS
EP = NW * EWP

_mesh = plsc.VectorSubcoreMesh(core_axis_name="c", subcore_axis_name="s")


@functools.partial(
    pl.kernel,
    out_type=jax.ShapeDtypeStruct((NC, NPAD, D), jnp.float32),
    mesh=_mesh,
    scratch_types=[
        pltpu.VMEM((JH * C,), jnp.int32),
        pltpu.VMEM((J, C), jnp.int32),
        pltpu.VMEM((C, D), jnp.float32),
        pltpu.VMEM((C, D), jnp.float32),
        pltpu.VMEM_SHARED((NPAD, D), jnp.float32),
        pltpu.SemaphoreType.DMA,
        pltpu.SemaphoreType.DMA,
    ],
)
def _sc_prop(u_hbm, col_hbm, row_hbm, zero_hbm, out_hbm,
             colv, rowv, b0, b1, acc, gsem, ssem):
    cid = lax.axis_index("c")
    sid = lax.axis_index("s")
    wid = cid * NS + sid

    pltpu.sync_copy(row_hbm.at[wid], rowv)
    pltpu.sync_copy(zero_hbm, acc.at[pl.ds(sid * RPT, RPT)])
    plsc.subcore_barrier()

    def g_start(jl, buf):
        pltpu.async_copy(u_hbm.at[colv.at[pl.ds(jl * C, C)]], buf, gsem)

    def g_wait(jl, buf):
        pltpu.make_async_copy(u_hbm.at[colv.at[pl.ds(jl * C, C)]], buf,
                              gsem).wait()

    def s_start(jg, buf):
        pltpu.async_copy(buf, acc.at[rowv.at[jg]], ssem, add=True)

    def s_wait(jg, buf):
        pltpu.make_async_copy(buf, acc.at[rowv.at[jg]], ssem).wait()

    for h in range(2):
        pltpu.sync_copy(col_hbm.at[wid, pl.ds(h * JH * C, JH * C)], colv)
        base = h * JH

        g_start(0, b0)
        g_start(1, b1)

        def body(i, carry, base=base):
            jl = i * 2
            g_wait(jl, b0)
            s_start(base + jl, b0)
            g_wait(jl + 1, b1)
            s_start(base + jl + 1, b1)
            s_wait(base + jl, b0)
            g_start(jl + 2, b0)
            s_wait(base + jl + 1, b1)
            g_start(jl + 3, b1)
            return carry

        lax.fori_loop(0, (JH - 2) // 2, body, 0)

        g_wait(JH - 2, b0)
        s_start(base + JH - 2, b0)
        g_wait(JH - 1, b1)
        s_start(base + JH - 1, b1)
        s_wait(base + JH - 2, b0)
        s_wait(base + JH - 1, b1)

    plsc.subcore_barrier()
    pltpu.sync_copy(acc.at[pl.ds(sid * RPT, RPT)],
                    out_hbm.at[cid, pl.ds(sid * RPT, RPT)])


BT = 2000
NB = N // BT


def _deg_body(sp_ref, x_ref, dinv_ref, u_ref):
    deg = sp_ref[0, :, 0:1] + sp_ref[1, :, 0:1]
    dinv = jnp.where(deg > 0, 1.0 / jnp.sqrt(jnp.maximum(deg, 1e-12)), 0.0)
    dinv_ref[...] = dinv
    u_ref[...] = dinv * x_ref[...]


def _tc_deg(sp, x):
    return pl.pallas_call(
        _deg_body,
        grid=(NB,),
        in_specs=[
            pl.BlockSpec((NC, BT, D), lambda i: (0, i, 0)),
            pl.BlockSpec((BT, D), lambda i: (i, 0)),
        ],
        out_specs=[
            pl.BlockSpec((BT, 1), lambda i: (i, 0)),
            pl.BlockSpec((BT, D), lambda i: (i, 0)),
        ],
        out_shape=[
            jax.ShapeDtypeStruct((N, 1), jnp.float32),
            jax.ShapeDtypeStruct((N, D), jnp.float32),
        ],
    )(sp, x)


def _mid_a_body(sp_ref, h_ref, dinv_ref, w_ref, b_ref, u_ref, p_ref):
    s = sp_ref[0] + sp_ref[1]
    dinv = dinv_ref[...]
    t = -dinv * s
    u_ref[...] = dinv * t
    w02 = w_ref[0] - w_ref[2]
    p_ref[...] = (
        jnp.dot(h_ref[...], w02, preferred_element_type=jnp.float32)
        + jnp.dot(t, w_ref[1], preferred_element_type=jnp.float32)
        + b_ref[...]
    )


def _tc_mid_a(sp, h, dinv, w, b):
    return pl.pallas_call(
        _mid_a_body,
        grid=(NB,),
        in_specs=[
            pl.BlockSpec((NC, BT, D), lambda i: (0, i, 0)),
            pl.BlockSpec((BT, D), lambda i: (i, 0)),
            pl.BlockSpec((BT, 1), lambda i: (i, 0)),
            pl.BlockSpec((3, D, D), lambda i: (0, 0, 0)),
            pl.BlockSpec((1, D), lambda i: (0, 0)),
        ],
        out_specs=[
            pl.BlockSpec((BT, D), lambda i: (i, 0)),
            pl.BlockSpec((BT, D), lambda i: (i, 0)),
        ],
        out_shape=[
            jax.ShapeDtypeStruct((N, D), jnp.float32),
            jax.ShapeDtypeStruct((N, D), jnp.float32),
        ],
    )(sp, h, dinv, w, b)


def _mid_b_body(sp_ref, p_ref, dinv_ref, w_ref, a_ref, h_ref, u_ref):
    s = sp_ref[0] + sp_ref[1]
    dinv = dinv_ref[...]
    q = dinv * s
    o = p_ref[...] - 2.0 * jnp.dot(q, w_ref[2],
                                   preferred_element_type=jnp.float32)
    hh = jnp.where(o >= 0, o, a_ref[...] * o)
    h_ref[...] = hh
    u_ref[...] = dinv * hh


def _tc_mid_b(sp, p, dinv, w, a):
    return pl.pallas_call(
        _mid_b_body,
        grid=(NB,),
        in_specs=[
            pl.BlockSpec((NC, BT, D), lambda i: (0, i, 0)),
            pl.BlockSpec((BT, D), lambda i: (i, 0)),
            pl.BlockSpec((BT, 1), lambda i: (i, 0)),
            pl.BlockSpec((3, D, D), lambda i: (0, 0, 0)),
            pl.BlockSpec((1, D), lambda i: (0, 0)),
        ],
        out_specs=[
            pl.BlockSpec((BT, D), lambda i: (i, 0)),
            pl.BlockSpec((BT, D), lambda i: (i, 0)),
        ],
        out_shape=[
            jax.ShapeDtypeStruct((N, D), jnp.float32),
            jax.ShapeDtypeStruct((N, D), jnp.float32),
        ],
    )(sp, p, dinv, w, a)


def _pool_body(h_ref, batch_ref, out_ref, cnt_acc):
    i = pl.program_id(0)
    onehot = (batch_ref[...] ==
              lax.broadcasted_iota(jnp.int32, (1, G), 1)).astype(jnp.float32)
    sums = lax.dot_general(onehot, h_ref[...], (((0,), (0,)), ((), ())),
                           preferred_element_type=jnp.float32)
    cnts = lax.dot_general(onehot, jnp.ones((BT, 1), jnp.float32),
                           (((0,), (0,)), ((), ())),
                           preferred_element_type=jnp.float32)

    @pl.when(i == 0)
    def _():
        out_ref[...] = jnp.zeros_like(out_ref)
        cnt_acc[...] = jnp.zeros_like(cnt_acc)

    out_ref[...] += sums
    cnt_acc[...] += cnts

    @pl.when(i == NB - 1)
    def _():
        out_ref[...] = out_ref[...] / jnp.maximum(cnt_acc[...], 1.0)


def _tc_pool(h, batch2):
    return pl.pallas_call(
        _pool_body,
        grid=(NB,),
        in_specs=[
            pl.BlockSpec((BT, D), lambda i: (i, 0)),
            pl.BlockSpec((BT, 1), lambda i: (i, 0)),
        ],
        out_specs=pl.BlockSpec((G, D), lambda i: (0, 0)),
        out_shape=jax.ShapeDtypeStruct((G, D), jnp.float32),
        scratch_shapes=[pltpu.VMEM((G, 1), jnp.float32)],
    )(h, batch2)


def kernel(x, edge_index, batch, W1, b1, a1, W2, b2, a2):
    row = edge_index[0].astype(jnp.int32)
    col = edge_index[1].astype(jnp.int32)
    npad = EP - E
    pad_ids = jnp.arange(npad, dtype=jnp.int32)
    rowp = jnp.concatenate([row, N + (pad_ids % (NPAD - N))])
    colp = jnp.concatenate([col, pad_ids % 128])
    row_h = rowp.reshape(NW, J, C)
    col_h = colp.reshape(NW, EWP)
    zero_rows = jnp.zeros((RPT, D), jnp.float32)
    ones_src = jnp.ones((N, D), jnp.float32)
    b1r = b1.reshape(1, D)
    a1r = a1.reshape(1, D)
    b2r = b2.reshape(1, D)
    a2r = a2.reshape(1, D)
    batch2 = batch.astype(jnp.int32).reshape(N, 1)

    def step(carry, i):
        u, p, h, dinv = carry
        sp = _sc_prop(u, col_h, row_h, zero_rows)

        def br_deg(_):
            dinv2, u0 = _tc_deg(sp, x)
            return (u0, p, h, dinv2)

        def br_mid1(_):
            u1, p1 = _tc_mid_a(sp, x, dinv, W1, b1r)
            return (u1, p1, h, dinv)

        def br_mid2(_):
            h1, u2 = _tc_mid_b(sp, p, dinv, W1, a1r)
            return (u2, p, h1, dinv)

        def br_mid3(_):
            u3, p2 = _tc_mid_a(sp, h, dinv, W2, b2r)
            return (u3, p2, h, dinv)

        def br_mid4(_):
            h2, u4 = _tc_mid_b(sp, p, dinv, W2, a2r)
            return (u4, p, h2, dinv)

        new_carry = lax.switch(i, [br_deg, br_mid1, br_mid2, br_mid3,
                                   br_mid4], None)
        return new_carry, None

    init = (ones_src, jnp.zeros((N, D), jnp.float32), x,
            jnp.zeros((N, 1), jnp.float32))
    (u_f, p_f, h_f, dinv_f), _ = lax.scan(
        step, init, jnp.arange(5, dtype=jnp.int32))
    return _tc_pool(h_f, batch2)


# speedup vs baseline: 14.9232x; 1.0037x over previous
import functools

import jax
import jax.numpy as jnp
from jax import lax
from jax.experimental import pallas as pl
from jax.experimental.pallas import tpu as pltpu
from jax.experimental.pallas import tpu_sc as plsc

N = 10000
E = 320000
D = 128
G = 16

NC, NS = 2, 16
NW = NC * NS
C = 128
EWP = 10240
J = EWP // C
JH = J // 2
NPAD = 10240
RPT = NPAD // NS
EP = NW * EWP

_mesh = plsc.VectorSubcoreMesh(core_axis_name="c", subcore_axis_name="s")


@functools.partial(
    pl.kernel,
    out_type=jax.ShapeDtypeStruct((NC, NPAD, D), jnp.float32),
    mesh=_mesh,
    scratch_types=[
        pltpu.VMEM((JH * C,), jnp.int32),
        pltpu.VMEM((J, C), jnp.int32),
        pltpu.VMEM((C, D), jnp.float32),
        pltpu.VMEM((C, D), jnp.float32),
        pltpu.VMEM_SHARED((NPAD, D), jnp.float32),
        pltpu.SemaphoreType.DMA,
        pltpu.SemaphoreType.DMA,
    ],
)
def _sc_prop(u_hbm, col_hbm, row_hbm, zero_hbm, out_hbm,
             colv, rowv, b0, b1, acc, gsem, ssem):
    cid = lax.axis_index("c")
    sid = lax.axis_index("s")
    wid = cid * NS + sid

    pltpu.sync_copy(row_hbm.at[wid], rowv)
    pltpu.sync_copy(zero_hbm, acc.at[pl.ds(sid * RPT, RPT)])
    plsc.subcore_barrier()

    def g_start(jl, buf):
        pltpu.async_copy(u_hbm.at[colv.at[pl.ds(jl * C, C)]], buf, gsem)

    def g_wait(jl, buf):
        pltpu.make_async_copy(u_hbm.at[colv.at[pl.ds(jl * C, C)]], buf,
                              gsem).wait()

    def s_start(jg, buf):
        pltpu.async_copy(buf, acc.at[rowv.at[jg]], ssem, add=True)

    def s_wait(jg, buf):
        pltpu.make_async_copy(buf, acc.at[rowv.at[jg]], ssem).wait()

    for h in range(2):
        pltpu.sync_copy(col_hbm.at[wid, pl.ds(h * JH * C, JH * C)], colv)
        base = h * JH

        g_start(0, b0)
        g_start(1, b1)

        def body(i, carry, base=base):
            jl = i * 2
            g_wait(jl, b0)
            s_start(base + jl, b0)
            g_wait(jl + 1, b1)
            s_start(base + jl + 1, b1)
            s_wait(base + jl, b0)
            g_start(jl + 2, b0)
            s_wait(base + jl + 1, b1)
            g_start(jl + 3, b1)
            return carry

        lax.fori_loop(0, (JH - 2) // 2, body, 0)

        g_wait(JH - 2, b0)
        s_start(base + JH - 2, b0)
        g_wait(JH - 1, b1)
        s_start(base + JH - 1, b1)
        s_wait(base + JH - 2, b0)
        s_wait(base + JH - 1, b1)

    plsc.subcore_barrier()
    pltpu.sync_copy(acc.at[pl.ds(sid * RPT, RPT)],
                    out_hbm.at[cid, pl.ds(sid * RPT, RPT)])


BT = 2000
NB = N // BT


def _deg_body(sp_ref, x_ref, dinv_ref, u_ref):
    deg = sp_ref[0, :, 0:1] + sp_ref[1, :, 0:1]
    dinv = jnp.where(deg > 0, 1.0 / jnp.sqrt(jnp.maximum(deg, 1e-12)), 0.0)
    dinv_ref[...] = dinv
    u_ref[...] = dinv * x_ref[...]


def _tc_deg(sp, x):
    return pl.pallas_call(
        _deg_body,
        grid=(NB,),
        in_specs=[
            pl.BlockSpec((NC, BT, D), lambda i: (0, i, 0)),
            pl.BlockSpec((BT, D), lambda i: (i, 0)),
        ],
        out_specs=[
            pl.BlockSpec((BT, 1), lambda i: (i, 0)),
            pl.BlockSpec((BT, D), lambda i: (i, 0)),
        ],
        out_shape=[
            jax.ShapeDtypeStruct((N, 1), jnp.float32),
            jax.ShapeDtypeStruct((N, D), jnp.float32),
        ],
    )(sp, x)


def _mid_a_body(sp_ref, h_ref, dinv_ref, w_ref, b_ref, u_ref, p_ref):
    s = sp_ref[0] + sp_ref[1]
    dinv = dinv_ref[...]
    t = -dinv * s
    u_ref[...] = dinv * t
    w02 = w_ref[0] - w_ref[2]
    p_ref[...] = (
        jnp.dot(h_ref[...], w02, preferred_element_type=jnp.float32)
        + jnp.dot(t, w_ref[1], preferred_element_type=jnp.float32)
        + b_ref[...]
    )


def _tc_mid_a(sp, h, dinv, w, b):
    return pl.pallas_call(
        _mid_a_body,
        grid=(NB,),
        in_specs=[
            pl.BlockSpec((NC, BT, D), lambda i: (0, i, 0)),
            pl.BlockSpec((BT, D), lambda i: (i, 0)),
            pl.BlockSpec((BT, 1), lambda i: (i, 0)),
            pl.BlockSpec((3, D, D), lambda i: (0, 0, 0)),
            pl.BlockSpec((1, D), lambda i: (0, 0)),
        ],
        out_specs=[
            pl.BlockSpec((BT, D), lambda i: (i, 0)),
            pl.BlockSpec((BT, D), lambda i: (i, 0)),
        ],
        out_shape=[
            jax.ShapeDtypeStruct((N, D), jnp.float32),
            jax.ShapeDtypeStruct((N, D), jnp.float32),
        ],
    )(sp, h, dinv, w, b)


def _mid_b_body(sp_ref, p_ref, dinv_ref, w_ref, a_ref, h_ref, u_ref):
    s = sp_ref[0] + sp_ref[1]
    dinv = dinv_ref[...]
    q = dinv * s
    o = p_ref[...] - 2.0 * jnp.dot(q, w_ref[2],
                                   preferred_element_type=jnp.float32)
    hh = jnp.where(o >= 0, o, a_ref[...] * o)
    h_ref[...] = hh
    u_ref[...] = dinv * hh


def _tc_mid_b(sp, p, dinv, w, a):
    return pl.pallas_call(
        _mid_b_body,
        grid=(NB,),
        in_specs=[
            pl.BlockSpec((NC, BT, D), lambda i: (0, i, 0)),
            pl.BlockSpec((BT, D), lambda i: (i, 0)),
            pl.BlockSpec((BT, 1), lambda i: (i, 0)),
            pl.BlockSpec((3, D, D), lambda i: (0, 0, 0)),
            pl.BlockSpec((1, D), lambda i: (0, 0)),
        ],
        out_specs=[
            pl.BlockSpec((BT, D), lambda i: (i, 0)),
            pl.BlockSpec((BT, D), lambda i: (i, 0)),
        ],
        out_shape=[
            jax.ShapeDtypeStruct((N, D), jnp.float32),
            jax.ShapeDtypeStruct((N, D), jnp.float32),
        ],
    )(sp, p, dinv, w, a)


def _mid_b_pool_body(sp_ref, p_ref, dinv_ref, w_ref, a_ref, batch_ref,
                     out_ref, cnt_acc):
    i = pl.program_id(0)
    s = sp_ref[0] + sp_ref[1]
    q = dinv_ref[...] * s
    o = p_ref[...] - 2.0 * jnp.dot(q, w_ref[2],
                                   preferred_element_type=jnp.float32)
    hh = jnp.where(o >= 0, o, a_ref[...] * o)
    onehot = (batch_ref[...] ==
              lax.broadcasted_iota(jnp.int32, (1, G), 1)).astype(jnp.float32)
    sums = lax.dot_general(onehot, hh, (((0,), (0,)), ((), ())),
                           preferred_element_type=jnp.float32)
    cnts = lax.dot_general(onehot, jnp.ones((BT, 1), jnp.float32),
                           (((0,), (0,)), ((), ())),
                           preferred_element_type=jnp.float32)

    @pl.when(i == 0)
    def _():
        out_ref[...] = jnp.zeros_like(out_ref)
        cnt_acc[...] = jnp.zeros_like(cnt_acc)

    out_ref[...] += sums
    cnt_acc[...] += cnts

    @pl.when(i == NB - 1)
    def _():
        out_ref[...] = out_ref[...] / jnp.maximum(cnt_acc[...], 1.0)


def _tc_mid_b_pool(sp, p, dinv, w, a, batch2):
    return pl.pallas_call(
        _mid_b_pool_body,
        grid=(NB,),
        in_specs=[
            pl.BlockSpec((NC, BT, D), lambda i: (0, i, 0)),
            pl.BlockSpec((BT, D), lambda i: (i, 0)),
            pl.BlockSpec((BT, 1), lambda i: (i, 0)),
            pl.BlockSpec((3, D, D), lambda i: (0, 0, 0)),
            pl.BlockSpec((1, D), lambda i: (0, 0)),
            pl.BlockSpec((BT, 1), lambda i: (i, 0)),
        ],
        out_specs=pl.BlockSpec((G, D), lambda i: (0, 0)),
        out_shape=jax.ShapeDtypeStruct((G, D), jnp.float32),
        scratch_shapes=[pltpu.VMEM((G, 1), jnp.float32)],
    )(sp, p, dinv, w, a, batch2)


def kernel(x, edge_index, batch, W1, b1, a1, W2, b2, a2):
    row = edge_index[0].astype(jnp.int32)
    col = edge_index[1].astype(jnp.int32)
    npad = EP - E
    pad_ids = jnp.arange(npad, dtype=jnp.int32)
    rowp = jnp.concatenate([row, N + (pad_ids % (NPAD - N))])
    colp = jnp.concatenate([col, pad_ids % 128])
    row_h = rowp.reshape(NW, J, C)
    col_h = colp.reshape(NW, EWP)
    zero_rows = jnp.zeros((RPT, D), jnp.float32)
    ones_src = jnp.ones((N, D), jnp.float32)
    b1r = b1.reshape(1, D)
    a1r = a1.reshape(1, D)
    b2r = b2.reshape(1, D)
    a2r = a2.reshape(1, D)
    batch2 = batch.astype(jnp.int32).reshape(N, 1)

    def step(carry, i):
        u, p, h, dinv, pool = carry
        sp = _sc_prop(u, col_h, row_h, zero_rows)

        def br_deg(_):
            dinv2, u0 = _tc_deg(sp, x)
            return (u0, p, h, dinv2, pool)

        def br_mid1(_):
            u1, p1 = _tc_mid_a(sp, x, dinv, W1, b1r)
            return (u1, p1, h, dinv, pool)

        def br_mid2(_):
            h1, u2 = _tc_mid_b(sp, p, dinv, W1, a1r)
            return (u2, p, h1, dinv, pool)

        def br_mid3(_):
            u3, p2 = _tc_mid_a(sp, h, dinv, W2, b2r)
            return (u3, p2, h, dinv, pool)

        def br_mid4(_):
            out = _tc_mid_b_pool(sp, p, dinv, W2, a2r, batch2)
            return (u, p, h, dinv, out)

        new_carry = lax.switch(i, [br_deg, br_mid1, br_mid2, br_mid3,
                                   br_mid4], None)
        return new_carry, None

    init = (ones_src, jnp.zeros((N, D), jnp.float32), x,
            jnp.zeros((N, 1), jnp.float32), jnp.zeros((G, D), jnp.float32))
    (_, _, _, _, pool_f), _ = lax.scan(
        step, init, jnp.arange(5, dtype=jnp.int32))
    return pool_f


# speedup vs baseline: 15.0744x; 1.0101x over previous
import functools

import jax
import jax.numpy as jnp
from jax import lax
from jax.experimental import pallas as pl
from jax.experimental.pallas import tpu as pltpu
from jax.experimental.pallas import tpu_sc as plsc

N = 10000
E = 320000
D = 128
G = 16

NC, NS = 2, 16
NW = NC * NS
C = 128
EWP = 10240
J = EWP // C
JH = J // 2
NPAD = 10240
RPT = NPAD // NS
EP = NW * EWP

_mesh = plsc.VectorSubcoreMesh(core_axis_name="c", subcore_axis_name="s")


@functools.partial(
    pl.kernel,
    out_type=jax.ShapeDtypeStruct((NC, NPAD, D), jnp.float32),
    mesh=_mesh,
    scratch_types=[
        pltpu.VMEM((JH * C,), jnp.int32),
        pltpu.VMEM((J, C), jnp.int32),
        pltpu.VMEM((C, D), jnp.float32),
        pltpu.VMEM((C, D), jnp.float32),
        pltpu.VMEM_SHARED((NPAD, D), jnp.float32),
        pltpu.SemaphoreType.DMA,
        pltpu.SemaphoreType.DMA,
    ],
)
def _sc_prop(u_hbm, col_hbm, row_hbm, zero_hbm, out_hbm,
             colv, rowv, b0, b1, acc, gsem, ssem):
    cid = lax.axis_index("c")
    sid = lax.axis_index("s")
    wid = cid * NS + sid

    pltpu.sync_copy(row_hbm.at[wid], rowv)
    pltpu.sync_copy(zero_hbm, acc.at[pl.ds(sid * RPT, RPT)])
    plsc.subcore_barrier()

    def g_start(jl, buf):
        pltpu.async_copy(u_hbm.at[colv.at[pl.ds(jl * C, C)]], buf, gsem)

    def g_wait(jl, buf):
        pltpu.make_async_copy(u_hbm.at[colv.at[pl.ds(jl * C, C)]], buf,
                              gsem).wait()

    def s_start(jg, buf):
        pltpu.async_copy(buf, acc.at[rowv.at[jg]], ssem, add=True)

    def s_wait(jg, buf):
        pltpu.make_async_copy(buf, acc.at[rowv.at[jg]], ssem).wait()

    for h in range(2):
        pltpu.sync_copy(col_hbm.at[wid, pl.ds(h * JH * C, JH * C)], colv)
        base = h * JH

        g_start(0, b0)
        g_start(1, b1)

        def body(i, carry, base=base):
            jl = i * 2
            g_wait(jl, b0)
            s_start(base + jl, b0)
            g_wait(jl + 1, b1)
            s_start(base + jl + 1, b1)
            s_wait(base + jl, b0)
            g_start(jl + 2, b0)
            s_wait(base + jl + 1, b1)
            g_start(jl + 3, b1)
            return carry

        lax.fori_loop(0, (JH - 2) // 2, body, 0)

        g_wait(JH - 2, b0)
        s_start(base + JH - 2, b0)
        g_wait(JH - 1, b1)
        s_start(base + JH - 1, b1)
        s_wait(base + JH - 2, b0)
        s_wait(base + JH - 1, b1)

    plsc.subcore_barrier()
    pltpu.sync_copy(acc.at[pl.ds(sid * RPT, RPT)],
                    out_hbm.at[cid, pl.ds(sid * RPT, RPT)])


BT = 5000
NB = N // BT


def _deg_body(sp_ref, x_ref, dinv_ref, u_ref):
    deg = sp_ref[0, :, 0:1] + sp_ref[1, :, 0:1]
    dinv = jnp.where(deg > 0, 1.0 / jnp.sqrt(jnp.maximum(deg, 1e-12)), 0.0)
    dinv_ref[...] = dinv
    u_ref[...] = dinv * x_ref[...]


def _tc_deg(sp, x):
    return pl.pallas_call(
        _deg_body,
        grid=(NB,),
        in_specs=[
            pl.BlockSpec((NC, BT, D), lambda i: (0, i, 0)),
            pl.BlockSpec((BT, D), lambda i: (i, 0)),
        ],
        out_specs=[
            pl.BlockSpec((BT, 1), lambda i: (i, 0)),
            pl.BlockSpec((BT, D), lambda i: (i, 0)),
        ],
        out_shape=[
            jax.ShapeDtypeStruct((N, 1), jnp.float32),
            jax.ShapeDtypeStruct((N, D), jnp.float32),
        ],
    )(sp, x)


def _mid_a_body(sp_ref, h_ref, dinv_ref, w_ref, b_ref, u_ref, p_ref):
    s = sp_ref[0] + sp_ref[1]
    dinv = dinv_ref[...]
    t = -dinv * s
    u_ref[...] = dinv * t
    w02 = w_ref[0] - w_ref[2]
    p_ref[...] = (
        jnp.dot(h_ref[...], w02, preferred_element_type=jnp.float32)
        + jnp.dot(t, w_ref[1], preferred_element_type=jnp.float32)
        + b_ref[...]
    )


def _tc_mid_a(sp, h, dinv, w, b):
    return pl.pallas_call(
        _mid_a_body,
        grid=(NB,),
        in_specs=[
            pl.BlockSpec((NC, BT, D), lambda i: (0, i, 0)),
            pl.BlockSpec((BT, D), lambda i: (i, 0)),
            pl.BlockSpec((BT, 1), lambda i: (i, 0)),
            pl.BlockSpec((3, D, D), lambda i: (0, 0, 0)),
            pl.BlockSpec((1, D), lambda i: (0, 0)),
        ],
        out_specs=[
            pl.BlockSpec((BT, D), lambda i: (i, 0)),
            pl.BlockSpec((BT, D), lambda i: (i, 0)),
        ],
        out_shape=[
            jax.ShapeDtypeStruct((N, D), jnp.float32),
            jax.ShapeDtypeStruct((N, D), jnp.float32),
        ],
    )(sp, h, dinv, w, b)


def _mid_b_body(sp_ref, p_ref, dinv_ref, w_ref, a_ref, h_ref, u_ref):
    s = sp_ref[0] + sp_ref[1]
    dinv = dinv_ref[...]
    q = dinv * s
    o = p_ref[...] - 2.0 * jnp.dot(q, w_ref[2],
                                   preferred_element_type=jnp.float32)
    hh = jnp.where(o >= 0, o, a_ref[...] * o)
    h_ref[...] = hh
    u_ref[...] = dinv * hh


def _tc_mid_b(sp, p, dinv, w, a):
    return pl.pallas_call(
        _mid_b_body,
        grid=(NB,),
        in_specs=[
            pl.BlockSpec((NC, BT, D), lambda i: (0, i, 0)),
            pl.BlockSpec((BT, D), lambda i: (i, 0)),
            pl.BlockSpec((BT, 1), lambda i: (i, 0)),
            pl.BlockSpec((3, D, D), lambda i: (0, 0, 0)),
            pl.BlockSpec((1, D), lambda i: (0, 0)),
        ],
        out_specs=[
            pl.BlockSpec((BT, D), lambda i: (i, 0)),
            pl.BlockSpec((BT, D), lambda i: (i, 0)),
        ],
        out_shape=[
            jax.ShapeDtypeStruct((N, D), jnp.float32),
            jax.ShapeDtypeStruct((N, D), jnp.float32),
        ],
    )(sp, p, dinv, w, a)


def _mid_b_pool_body(sp_ref, p_ref, dinv_ref, w_ref, a_ref, batch_ref,
                     out_ref, cnt_acc):
    i = pl.program_id(0)
    s = sp_ref[0] + sp_ref[1]
    q = dinv_ref[...] * s
    o = p_ref[...] - 2.0 * jnp.dot(q, w_ref[2],
                                   preferred_element_type=jnp.float32)
    hh = jnp.where(o >= 0, o, a_ref[...] * o)
    onehot = (batch_ref[...] ==
              lax.broadcasted_iota(jnp.int32, (1, G), 1)).astype(jnp.float32)
    sums = lax.dot_general(onehot, hh, (((0,), (0,)), ((), ())),
                           preferred_element_type=jnp.float32)
    cnts = lax.dot_general(onehot, jnp.ones((BT, 1), jnp.float32),
                           (((0,), (0,)), ((), ())),
                           preferred_element_type=jnp.float32)

    @pl.when(i == 0)
    def _():
        out_ref[...] = jnp.zeros_like(out_ref)
        cnt_acc[...] = jnp.zeros_like(cnt_acc)

    out_ref[...] += sums
    cnt_acc[...] += cnts

    @pl.when(i == NB - 1)
    def _():
        out_ref[...] = out_ref[...] / jnp.maximum(cnt_acc[...], 1.0)


def _tc_mid_b_pool(sp, p, dinv, w, a, batch2):
    return pl.pallas_call(
        _mid_b_pool_body,
        grid=(NB,),
        in_specs=[
            pl.BlockSpec((NC, BT, D), lambda i: (0, i, 0)),
            pl.BlockSpec((BT, D), lambda i: (i, 0)),
            pl.BlockSpec((BT, 1), lambda i: (i, 0)),
            pl.BlockSpec((3, D, D), lambda i: (0, 0, 0)),
            pl.BlockSpec((1, D), lambda i: (0, 0)),
            pl.BlockSpec((BT, 1), lambda i: (i, 0)),
        ],
        out_specs=pl.BlockSpec((G, D), lambda i: (0, 0)),
        out_shape=jax.ShapeDtypeStruct((G, D), jnp.float32),
        scratch_shapes=[pltpu.VMEM((G, 1), jnp.float32)],
    )(sp, p, dinv, w, a, batch2)


def kernel(x, edge_index, batch, W1, b1, a1, W2, b2, a2):
    row = edge_index[0].astype(jnp.int32)
    col = edge_index[1].astype(jnp.int32)
    npad = EP - E
    pad_ids = jnp.arange(npad, dtype=jnp.int32)
    rowp = jnp.concatenate([row, N + (pad_ids % (NPAD - N))])
    colp = jnp.concatenate([col, pad_ids % 128])
    row_h = rowp.reshape(NW, J, C)
    col_h = colp.reshape(NW, EWP)
    zero_rows = jnp.zeros((RPT, D), jnp.float32)
    ones_src = jnp.ones((N, D), jnp.float32)
    b1r = b1.reshape(1, D)
    a1r = a1.reshape(1, D)
    b2r = b2.reshape(1, D)
    a2r = a2.reshape(1, D)
    batch2 = batch.astype(jnp.int32).reshape(N, 1)

    def step(carry, i):
        u, p, h, dinv, pool = carry
        sp = _sc_prop(u, col_h, row_h, zero_rows)

        def br_deg(_):
            dinv2, u0 = _tc_deg(sp, x)
            return (u0, p, h, dinv2, pool)

        def br_mid1(_):
            u1, p1 = _tc_mid_a(sp, x, dinv, W1, b1r)
            return (u1, p1, h, dinv, pool)

        def br_mid2(_):
            h1, u2 = _tc_mid_b(sp, p, dinv, W1, a1r)
            return (u2, p, h1, dinv, pool)

        def br_mid3(_):
            u3, p2 = _tc_mid_a(sp, h, dinv, W2, b2r)
            return (u3, p2, h, dinv, pool)

        def br_mid4(_):
            out = _tc_mid_b_pool(sp, p, dinv, W2, a2r, batch2)
            return (u, p, h, dinv, out)

        new_carry = lax.switch(i, [br_deg, br_mid1, br_mid2, br_mid3,
                                   br_mid4], None)
        return new_carry, None

    init = (ones_src, jnp.zeros((N, D), jnp.float32), x,
            jnp.zeros((N, 1), jnp.float32), jnp.zeros((G, D), jnp.float32))
    (_, _, _, _, pool_f), _ = lax.scan(
        step, init, jnp.arange(5, dtype=jnp.int32))
    return pool_f


# speedup vs baseline: 15.3204x; 1.0163x over previous
import functools

import jax
import jax.numpy as jnp
from jax import lax
from jax.experimental import pallas as pl
from jax.experimental.pallas import tpu as pltpu
from jax.experimental.pallas import tpu_sc as plsc

N = 10000
E = 320000
D = 128
G = 16

NC, NS = 2, 16
NW = NC * NS
C = 128
EWP = 10240
J = EWP // C
JH = J // 2
NPAD = 10240
RPT = NPAD // NS
EP = NW * EWP

_mesh = plsc.VectorSubcoreMesh(core_axis_name="c", subcore_axis_name="s")


@functools.partial(
    pl.kernel,
    out_type=jax.ShapeDtypeStruct((NC, NPAD, D), jnp.float32),
    mesh=_mesh,
    scratch_types=[
        pltpu.VMEM((JH * C,), jnp.int32),
        pltpu.VMEM((J, C), jnp.int32),
        pltpu.VMEM((C, D), jnp.float32),
        pltpu.VMEM((C, D), jnp.float32),
        pltpu.VMEM_SHARED((NPAD, D), jnp.float32),
        pltpu.SemaphoreType.DMA,
        pltpu.SemaphoreType.DMA,
    ],
)
def _sc_prop(u_hbm, col_hbm, row_hbm, zero_hbm, out_hbm,
             colv, rowv, b0, b1, acc, gsem, ssem):
    cid = lax.axis_index("c")
    sid = lax.axis_index("s")
    wid = cid * NS + sid

    def g_start(jl, buf):
        pltpu.async_copy(u_hbm.at[colv.at[pl.ds(jl * C, C)]], buf, gsem)

    def g_wait(jl, buf):
        pltpu.make_async_copy(u_hbm.at[colv.at[pl.ds(jl * C, C)]], buf,
                              gsem).wait()

    def s_start(jg, buf):
        pltpu.async_copy(buf, acc.at[rowv.at[jg]], ssem, add=True)

    def s_wait(jg, buf):
        pltpu.make_async_copy(buf, acc.at[rowv.at[jg]], ssem).wait()

    pltpu.async_copy(row_hbm.at[wid], rowv, ssem)
    pltpu.async_copy(col_hbm.at[wid, pl.ds(0, JH * C)], colv, ssem)
    pltpu.make_async_copy(row_hbm.at[wid], rowv, ssem).wait()
    pltpu.make_async_copy(col_hbm.at[wid, pl.ds(0, JH * C)], colv,
                          ssem).wait()
    g_start(0, b0)
    g_start(1, b1)
    pltpu.sync_copy(zero_hbm, acc.at[pl.ds(sid * RPT, RPT)])
    plsc.subcore_barrier()

    for h in range(2):
        if h == 1:
            pltpu.sync_copy(col_hbm.at[wid, pl.ds(JH * C, JH * C)], colv)
            g_start(0, b0)
            g_start(1, b1)
        base = h * JH

        def body(i, carry, base=base):
            jl = i * 2
            g_wait(jl, b0)
            s_start(base + jl, b0)
            g_wait(jl + 1, b1)
            s_start(base + jl + 1, b1)
            s_wait(base + jl, b0)
            g_start(jl + 2, b0)
            s_wait(base + jl + 1, b1)
            g_start(jl + 3, b1)
            return carry

        lax.fori_loop(0, (JH - 2) // 2, body, 0)

        g_wait(JH - 2, b0)
        s_start(base + JH - 2, b0)
        g_wait(JH - 1, b1)
        s_start(base + JH - 1, b1)
        s_wait(base + JH - 2, b0)
        s_wait(base + JH - 1, b1)

    plsc.subcore_barrier()
    pltpu.sync_copy(acc.at[pl.ds(sid * RPT, RPT)],
                    out_hbm.at[cid, pl.ds(sid * RPT, RPT)])


BT = 5000
NB = N // BT


def _deg_body(sp_ref, x_ref, dinv_ref, u_ref):
    deg = sp_ref[0, :, 0:1] + sp_ref[1, :, 0:1]
    dinv = jnp.where(deg > 0, 1.0 / jnp.sqrt(jnp.maximum(deg, 1e-12)), 0.0)
    dinv_ref[...] = dinv
    u_ref[...] = dinv * x_ref[...]


def _tc_deg(sp, x):
    return pl.pallas_call(
        _deg_body,
        grid=(NB,),
        in_specs=[
            pl.BlockSpec((NC, BT, D), lambda i: (0, i, 0)),
            pl.BlockSpec((BT, D), lambda i: (i, 0)),
        ],
        out_specs=[
            pl.BlockSpec((BT, 1), lambda i: (i, 0)),
            pl.BlockSpec((BT, D), lambda i: (i, 0)),
        ],
        out_shape=[
            jax.ShapeDtypeStruct((N, 1), jnp.float32),
            jax.ShapeDtypeStruct((N, D), jnp.float32),
        ],
    )(sp, x)


def _mid_a_body(sp_ref, h_ref, dinv_ref, w_ref, b_ref, u_ref, p_ref):
    s = sp_ref[0] + sp_ref[1]
    dinv = dinv_ref[...]
    t = -dinv * s
    u_ref[...] = dinv * t
    w02 = w_ref[0] - w_ref[2]
    p_ref[...] = (
        jnp.dot(h_ref[...], w02, preferred_element_type=jnp.float32)
        + jnp.dot(t, w_ref[1], preferred_element_type=jnp.float32)
        + b_ref[...]
    )


def _tc_mid_a(sp, h, dinv, w, b):
    return pl.pallas_call(
        _mid_a_body,
        grid=(NB,),
        in_specs=[
            pl.BlockSpec((NC, BT, D), lambda i: (0, i, 0)),
            pl.BlockSpec((BT, D), lambda i: (i, 0)),
            pl.BlockSpec((BT, 1), lambda i: (i, 0)),
            pl.BlockSpec((3, D, D), lambda i: (0, 0, 0)),
            pl.BlockSpec((1, D), lambda i: (0, 0)),
        ],
        out_specs=[
            pl.BlockSpec((BT, D), lambda i: (i, 0)),
            pl.BlockSpec((BT, D), lambda i: (i, 0)),
        ],
        out_shape=[
            jax.ShapeDtypeStruct((N, D), jnp.float32),
            jax.ShapeDtypeStruct((N, D), jnp.float32),
        ],
    )(sp, h, dinv, w, b)


def _mid_b_body(sp_ref, p_ref, dinv_ref, w_ref, a_ref, h_ref, u_ref):
    s = sp_ref[0] + sp_ref[1]
    dinv = dinv_ref[...]
    q = dinv * s
    o = p_ref[...] - 2.0 * jnp.dot(q, w_ref[2],
                                   preferred_element_type=jnp.float32)
    hh = jnp.where(o >= 0, o, a_ref[...] * o)
    h_ref[...] = hh
    u_ref[...] = dinv * hh


def _tc_mid_b(sp, p, dinv, w, a):
    return pl.pallas_call(
        _mid_b_body,
        grid=(NB,),
        in_specs=[
            pl.BlockSpec((NC, BT, D), lambda i: (0, i, 0)),
            pl.BlockSpec((BT, D), lambda i: (i, 0)),
            pl.BlockSpec((BT, 1), lambda i: (i, 0)),
            pl.BlockSpec((3, D, D), lambda i: (0, 0, 0)),
            pl.BlockSpec((1, D), lambda i: (0, 0)),
        ],
        out_specs=[
            pl.BlockSpec((BT, D), lambda i: (i, 0)),
            pl.BlockSpec((BT, D), lambda i: (i, 0)),
        ],
        out_shape=[
            jax.ShapeDtypeStruct((N, D), jnp.float32),
            jax.ShapeDtypeStruct((N, D), jnp.float32),
        ],
    )(sp, p, dinv, w, a)


def _mid_b_pool_body(sp_ref, p_ref, dinv_ref, w_ref, a_ref, batch_ref,
                     out_ref, cnt_acc):
    i = pl.program_id(0)
    s = sp_ref[0] + sp_ref[1]
    q = dinv_ref[...] * s
    o = p_ref[...] - 2.0 * jnp.dot(q, w_ref[2],
                                   preferred_element_type=jnp.float32)
    hh = jnp.where(o >= 0, o, a_ref[...] * o)
    onehot = (batch_ref[...] ==
              lax.broadcasted_iota(jnp.int32, (1, G), 1)).astype(jnp.float32)
    sums = lax.dot_general(onehot, hh, (((0,), (0,)), ((), ())),
                           preferred_element_type=jnp.float32)
    cnts = lax.dot_general(onehot, jnp.ones((BT, 1), jnp.float32),
                           (((0,), (0,)), ((), ())),
                           preferred_element_type=jnp.float32)

    @pl.when(i == 0)
    def _():
        out_ref[...] = jnp.zeros_like(out_ref)
        cnt_acc[...] = jnp.zeros_like(cnt_acc)

    out_ref[...] += sums
    cnt_acc[...] += cnts

    @pl.when(i == NB - 1)
    def _():
        out_ref[...] = out_ref[...] / jnp.maximum(cnt_acc[...], 1.0)


def _tc_mid_b_pool(sp, p, dinv, w, a, batch2):
    return pl.pallas_call(
        _mid_b_pool_body,
        grid=(NB,),
        in_specs=[
            pl.BlockSpec((NC, BT, D), lambda i: (0, i, 0)),
            pl.BlockSpec((BT, D), lambda i: (i, 0)),
            pl.BlockSpec((BT, 1), lambda i: (i, 0)),
            pl.BlockSpec((3, D, D), lambda i: (0, 0, 0)),
            pl.BlockSpec((1, D), lambda i: (0, 0)),
            pl.BlockSpec((BT, 1), lambda i: (i, 0)),
        ],
        out_specs=pl.BlockSpec((G, D), lambda i: (0, 0)),
        out_shape=jax.ShapeDtypeStruct((G, D), jnp.float32),
        scratch_shapes=[pltpu.VMEM((G, 1), jnp.float32)],
    )(sp, p, dinv, w, a, batch2)


def kernel(x, edge_index, batch, W1, b1, a1, W2, b2, a2):
    row = edge_index[0].astype(jnp.int32)
    col = edge_index[1].astype(jnp.int32)
    npad = EP - E
    pad_ids = jnp.arange(npad, dtype=jnp.int32)
    rowp = jnp.concatenate([row, N + (pad_ids % (NPAD - N))])
    colp = jnp.concatenate([col, pad_ids % 128])
    row_h = rowp.reshape(NW, J, C)
    col_h = colp.reshape(NW, EWP)
    zero_rows = jnp.zeros((RPT, D), jnp.float32)
    ones_src = jnp.ones((N, D), jnp.float32)
    b1r = b1.reshape(1, D)
    a1r = a1.reshape(1, D)
    b2r = b2.reshape(1, D)
    a2r = a2.reshape(1, D)
    batch2 = batch.astype(jnp.int32).reshape(N, 1)

    def step(carry, i):
        u, p, h, dinv, pool = carry
        sp = _sc_prop(u, col_h, row_h, zero_rows)

        def br_deg(_):
            dinv2, u0 = _tc_deg(sp, x)
            return (u0, p, h, dinv2, pool)

        def br_mid1(_):
            u1, p1 = _tc_mid_a(sp, x, dinv, W1, b1r)
            return (u1, p1, h, dinv, pool)

        def br_mid2(_):
            h1, u2 = _tc_mid_b(sp, p, dinv, W1, a1r)
            return (u2, p, h1, dinv, pool)

        def br_mid3(_):
            u3, p2 = _tc_mid_a(sp, h, dinv, W2, b2r)
            return (u3, p2, h, dinv, pool)

        def br_mid4(_):
            out = _tc_mid_b_pool(sp, p, dinv, W2, a2r, batch2)
            return (u, p, h, dinv, out)

        new_carry = lax.switch(i, [br_deg, br_mid1, br_mid2, br_mid3,
                                   br_mid4], None)
        return new_carry, None

    init = (ones_src, jnp.zeros((N, D), jnp.float32), x,
            jnp.zeros((N, 1), jnp.float32), jnp.zeros((G, D), jnp.float32))
    (_, _, _, _, pool_f), _ = lax.scan(
        step, init, jnp.arange(5, dtype=jnp.int32))
    return pool_f


# speedup vs baseline: 15.3627x; 1.0028x over previous
import functools

import jax
import jax.numpy as jnp
from jax import lax
from jax.experimental import pallas as pl
from jax.experimental.pallas import tpu as pltpu
from jax.experimental.pallas import tpu_sc as plsc

N = 10000
E = 320000
D = 128
G = 16

NC, NS = 2, 16
NW = NC * NS
C = 128
EWP = 10240
J = EWP // C
JH = J // 2
NPAD = 10240
RPT = NPAD // NS
EP = NW * EWP

_mesh = plsc.VectorSubcoreMesh(core_axis_name="c", subcore_axis_name="s")


@functools.partial(
    pl.kernel,
    out_type=jax.ShapeDtypeStruct((NC, NPAD, D), jnp.float32),
    mesh=_mesh,
    scratch_types=[
        pltpu.VMEM((JH * C,), jnp.int32),
        pltpu.VMEM((J, C), jnp.int32),
        pltpu.VMEM((C, D), jnp.float32),
        pltpu.VMEM((C, D), jnp.float32),
        pltpu.VMEM_SHARED((NPAD, D), jnp.float32),
        pltpu.SemaphoreType.DMA,
        pltpu.SemaphoreType.DMA,
    ],
)
def _sc_prop(u_hbm, col_hbm, row_hbm, zero_hbm, out_hbm,
             colv, rowv, b0, b1, acc, gsem, ssem):
    cid = lax.axis_index("c")
    sid = lax.axis_index("s")
    wid = cid * NS + sid

    def g_start(jl, buf):
        pltpu.async_copy(u_hbm.at[colv.at[pl.ds(jl * C, C)]], buf, gsem)

    def g_wait(jl, buf):
        pltpu.make_async_copy(u_hbm.at[colv.at[pl.ds(jl * C, C)]], buf,
                              gsem).wait()

    def s_start(jg, buf):
        pltpu.async_copy(buf, acc.at[rowv.at[jg]], ssem, add=True)

    def s_wait(jg, buf):
        pltpu.make_async_copy(buf, acc.at[rowv.at[jg]], ssem).wait()

    pltpu.async_copy(row_hbm.at[wid], rowv, ssem)
    pltpu.async_copy(col_hbm.at[wid, pl.ds(0, JH * C)], colv, ssem)
    pltpu.make_async_copy(row_hbm.at[wid], rowv, ssem).wait()
    pltpu.make_async_copy(col_hbm.at[wid, pl.ds(0, JH * C)], colv,
                          ssem).wait()
    g_start(0, b0)
    g_start(1, b1)
    pltpu.sync_copy(zero_hbm, acc.at[pl.ds(sid * RPT, RPT)])
    plsc.subcore_barrier()

    for h in range(2):
        base = h * JH

        def body(i, carry, base=base):
            jl = i * 2
            g_wait(jl, b0)
            s_start(base + jl, b0)
            g_wait(jl + 1, b1)
            s_start(base + jl + 1, b1)
            s_wait(base + jl, b0)
            g_start(jl + 2, b0)
            s_wait(base + jl + 1, b1)
            g_start(jl + 3, b1)
            return carry

        lax.fori_loop(0, (JH - 2) // 2, body, 0)

        g_wait(JH - 2, b0)
        s_start(base + JH - 2, b0)
        g_wait(JH - 1, b1)
        s_start(base + JH - 1, b1)
        if h == 0:
            pltpu.sync_copy(col_hbm.at[wid, pl.ds(JH * C, JH * C)], colv)
            s_wait(base + JH - 2, b0)
            g_start(0, b0)
            s_wait(base + JH - 1, b1)
            g_start(1, b1)
        else:
            s_wait(base + JH - 2, b0)
            s_wait(base + JH - 1, b1)

    plsc.subcore_barrier()
    pltpu.sync_copy(acc.at[pl.ds(sid * RPT, RPT)],
                    out_hbm.at[cid, pl.ds(sid * RPT, RPT)])


BT = 5000
NB = N // BT


def _deg_body(sp_ref, x_ref, dinv_ref, u_ref):
    deg = sp_ref[0, :, 0:1] + sp_ref[1, :, 0:1]
    dinv = jnp.where(deg > 0, 1.0 / jnp.sqrt(jnp.maximum(deg, 1e-12)), 0.0)
    dinv_ref[...] = dinv
    u_ref[...] = dinv * x_ref[...]


def _tc_deg(sp, x):
    return pl.pallas_call(
        _deg_body,
        grid=(NB,),
        in_specs=[
            pl.BlockSpec((NC, BT, D), lambda i: (0, i, 0)),
            pl.BlockSpec((BT, D), lambda i: (i, 0)),
        ],
        out_specs=[
            pl.BlockSpec((BT, 1), lambda i: (i, 0)),
            pl.BlockSpec((BT, D), lambda i: (i, 0)),
        ],
        out_shape=[
            jax.ShapeDtypeStruct((N, 1), jnp.float32),
            jax.ShapeDtypeStruct((N, D), jnp.float32),
        ],
    )(sp, x)


def _mid_a_body(sp_ref, h_ref, dinv_ref, w_ref, b_ref, u_ref, p_ref):
    s = sp_ref[0] + sp_ref[1]
    dinv = dinv_ref[...]
    t = -dinv * s
    u_ref[...] = dinv * t
    w02 = w_ref[0] - w_ref[2]
    p_ref[...] = (
        jnp.dot(h_ref[...], w02, preferred_element_type=jnp.float32)
        + jnp.dot(t, w_ref[1], preferred_element_type=jnp.float32)
        + b_ref[...]
    )


def _tc_mid_a(sp, h, dinv, w, b):
    return pl.pallas_call(
        _mid_a_body,
        grid=(NB,),
        in_specs=[
            pl.BlockSpec((NC, BT, D), lambda i: (0, i, 0)),
            pl.BlockSpec((BT, D), lambda i: (i, 0)),
            pl.BlockSpec((BT, 1), lambda i: (i, 0)),
            pl.BlockSpec((3, D, D), lambda i: (0, 0, 0)),
            pl.BlockSpec((1, D), lambda i: (0, 0)),
        ],
        out_specs=[
            pl.BlockSpec((BT, D), lambda i: (i, 0)),
            pl.BlockSpec((BT, D), lambda i: (i, 0)),
        ],
        out_shape=[
            jax.ShapeDtypeStruct((N, D), jnp.float32),
            jax.ShapeDtypeStruct((N, D), jnp.float32),
        ],
    )(sp, h, dinv, w, b)


def _mid_b_body(sp_ref, p_ref, dinv_ref, w_ref, a_ref, h_ref, u_ref):
    s = sp_ref[0] + sp_ref[1]
    dinv = dinv_ref[...]
    q = dinv * s
    o = p_ref[...] - 2.0 * jnp.dot(q, w_ref[2],
                                   preferred_element_type=jnp.float32)
    hh = jnp.where(o >= 0, o, a_ref[...] * o)
    h_ref[...] = hh
    u_ref[...] = dinv * hh


def _tc_mid_b(sp, p, dinv, w, a):
    return pl.pallas_call(
        _mid_b_body,
        grid=(NB,),
        in_specs=[
            pl.BlockSpec((NC, BT, D), lambda i: (0, i, 0)),
            pl.BlockSpec((BT, D), lambda i: (i, 0)),
            pl.BlockSpec((BT, 1), lambda i: (i, 0)),
            pl.BlockSpec((3, D, D), lambda i: (0, 0, 0)),
            pl.BlockSpec((1, D), lambda i: (0, 0)),
        ],
        out_specs=[
            pl.BlockSpec((BT, D), lambda i: (i, 0)),
            pl.BlockSpec((BT, D), lambda i: (i, 0)),
        ],
        out_shape=[
            jax.ShapeDtypeStruct((N, D), jnp.float32),
            jax.ShapeDtypeStruct((N, D), jnp.float32),
        ],
    )(sp, p, dinv, w, a)


def _mid_b_pool_body(sp_ref, p_ref, dinv_ref, w_ref, a_ref, batch_ref,
                     out_ref, cnt_acc):
    i = pl.program_id(0)
    s = sp_ref[0] + sp_ref[1]
    q = dinv_ref[...] * s
    o = p_ref[...] - 2.0 * jnp.dot(q, w_ref[2],
                                   preferred_element_type=jnp.float32)
    hh = jnp.where(o >= 0, o, a_ref[...] * o)
    onehot = (batch_ref[...] ==
              lax.broadcasted_iota(jnp.int32, (1, G), 1)).astype(jnp.float32)
    sums = lax.dot_general(onehot, hh, (((0,), (0,)), ((), ())),
                           preferred_element_type=jnp.float32)
    cnts = lax.dot_general(onehot, jnp.ones((BT, 1), jnp.float32),
                           (((0,), (0,)), ((), ())),
                           preferred_element_type=jnp.float32)

    @pl.when(i == 0)
    def _():
        out_ref[...] = jnp.zeros_like(out_ref)
        cnt_acc[...] = jnp.zeros_like(cnt_acc)

    out_ref[...] += sums
    cnt_acc[...] += cnts

    @pl.when(i == NB - 1)
    def _():
        out_ref[...] = out_ref[...] / jnp.maximum(cnt_acc[...], 1.0)


def _tc_mid_b_pool(sp, p, dinv, w, a, batch2):
    return pl.pallas_call(
        _mid_b_pool_body,
        grid=(NB,),
        in_specs=[
            pl.BlockSpec((NC, BT, D), lambda i: (0, i, 0)),
            pl.BlockSpec((BT, D), lambda i: (i, 0)),
            pl.BlockSpec((BT, 1), lambda i: (i, 0)),
            pl.BlockSpec((3, D, D), lambda i: (0, 0, 0)),
            pl.BlockSpec((1, D), lambda i: (0, 0)),
            pl.BlockSpec((BT, 1), lambda i: (i, 0)),
        ],
        out_specs=pl.BlockSpec((G, D), lambda i: (0, 0)),
        out_shape=jax.ShapeDtypeStruct((G, D), jnp.float32),
        scratch_shapes=[pltpu.VMEM((G, 1), jnp.float32)],
    )(sp, p, dinv, w, a, batch2)


def kernel(x, edge_index, batch, W1, b1, a1, W2, b2, a2):
    row = edge_index[0].astype(jnp.int32)
    col = edge_index[1].astype(jnp.int32)
    npad = EP - E
    pad_ids = jnp.arange(npad, dtype=jnp.int32)
    rowp = jnp.concatenate([row, N + (pad_ids % (NPAD - N))])
    colp = jnp.concatenate([col, pad_ids % 128])
    row_h = rowp.reshape(NW, J, C)
    col_h = colp.reshape(NW, EWP)
    zero_rows = jnp.zeros((RPT, D), jnp.float32)
    ones_src = jnp.ones((N, D), jnp.float32)
    b1r = b1.reshape(1, D)
    a1r = a1.reshape(1, D)
    b2r = b2.reshape(1, D)
    a2r = a2.reshape(1, D)
    batch2 = batch.astype(jnp.int32).reshape(N, 1)

    def step(carry, i):
        u, p, h, dinv, pool = carry
        sp = _sc_prop(u, col_h, row_h, zero_rows)

        def br_deg(_):
            dinv2, u0 = _tc_deg(sp, x)
            return (u0, p, h, dinv2, pool)

        def br_mid1(_):
            u1, p1 = _tc_mid_a(sp, x, dinv, W1, b1r)
            return (u1, p1, h, dinv, pool)

        def br_mid2(_):
            h1, u2 = _tc_mid_b(sp, p, dinv, W1, a1r)
            return (u2, p, h1, dinv, pool)

        def br_mid3(_):
            u3, p2 = _tc_mid_a(sp, h, dinv, W2, b2r)
            return (u3, p2, h, dinv, pool)

        def br_mid4(_):
            out = _tc_mid_b_pool(sp, p, dinv, W2, a2r, batch2)
            return (u, p, h, dinv, out)

        new_carry = lax.switch(i, [br_deg, br_mid1, br_mid2, br_mid3,
                                   br_mid4], None)
        return new_carry, None

    init = (ones_src, jnp.zeros((N, D), jnp.float32), x,
            jnp.zeros((N, 1), jnp.float32), jnp.zeros((G, D), jnp.float32))
    (_, _, _, _, pool_f), _ = lax.scan(
        step, init, jnp.arange(5, dtype=jnp.int32))
    return pool_f
